# baseline (speedup 1.0000x reference)
import functools

import jax
import jax.numpy as jnp
from jax import lax
from jax.experimental import pallas as pl
from jax.experimental.pallas import tpu as pltpu
from jax.experimental.pallas import tpu_sc as plsc

_NC = 2
_NS = 16
_L = 16
_NW = _NC * _NS

_CH = 112
_DL = 128
_PP = 64


def _stage1_body(S, S_PAD, nx, ny, nz, tv_ref, p_ref, mb_ref,
                 sidx_ref, rid_ref, w_ref, nc_ref, nsl_ref):
    tv = tv_ref[...]
    p = p_ref[...]
    t0 = tv[:, :-1]
    t1 = tv[:, 1:]
    fin = jnp.isfinite(t0) & jnp.isfinite(t1)
    t0s = jnp.where(fin, t0, 0.0)
    t1s = jnp.where(fin, t1, 0.0)
    valid = fin & (t1 > t0)
    tmid = 0.5 * (t0s + t1s)
    seg = (t1s - t0s) * p[:, 6:7]
    def q(x):
        return x.astype(jnp.bfloat16).astype(jnp.float32)
    px = q(p[:, 0:1] + tmid * p[:, 3:4])
    py = q(p[:, 1:2] + tmid * p[:, 4:5])
    pz = q(p[:, 2:3] + tmid * p[:, 5:6])
    vx = px * mb_ref[0] + py * mb_ref[1] + pz * mb_ref[2] + mb_ref[9]
    vy = px * mb_ref[3] + py * mb_ref[4] + pz * mb_ref[5] + mb_ref[10]
    vz = px * mb_ref[6] + py * mb_ref[7] + pz * mb_ref[8] + mb_ref[11]
    ix = jnp.floor(vx).astype(jnp.int32)
    iy = jnp.floor(vy).astype(jnp.int32)
    iz = jnp.floor(vz).astype(jnp.int32)
    inb = ((ix >= 0) & (ix < nx) & (iy >= 0) & (iy < ny)
           & (iz >= 0) & (iz < nz))
    ixc = jnp.clip(ix, 0, nx - 1)
    iyc = jnp.clip(iy, 0, ny - 1)
    izc = jnp.clip(iz, 0, nz - 1)
    w = jnp.where(valid & inb, seg, 0.0)
    rb = tv.shape[0]
    row = (izc * ny + iyc) * (nx // _DL) + (ixc // _DL)
    lane = ixc & (_DL - 1)
    prevrow = jnp.concatenate([row[:, :1], row[:, :-1]], axis=1)
    colid = lax.broadcasted_iota(jnp.int32, row.shape, 1)
    chg = (fin & (row != prevrow) & (colid > 0)).astype(jnp.int32)
    x = chg
    sh = 1
    while sh < S:
        x = x + jnp.concatenate(
            [jnp.zeros((rb, sh), jnp.int32), x[:, :-sh]], axis=1)
        sh *= 2
    slot = x
    nseg = jnp.sum(fin.astype(jnp.int32), axis=1, keepdims=True)
    nsl_ref[...] = jnp.where(nseg > 0, x[:, -1:] + 1, 0)
    trash = _PP * _DL
    sidx = jnp.where(w > 0, slot * _DL + lane, trash + lane)
    pad = S_PAD - S
    sidx_ref[...] = jnp.concatenate(
        [sidx, jnp.full((rb, pad), trash, jnp.int32)], axis=1)
    rid_ref[...] = jnp.concatenate(
        [row, jnp.zeros((rb, pad), jnp.int32)], axis=1)
    w_ref[...] = jnp.concatenate(
        [w, jnp.zeros((rb, pad), jnp.float32)], axis=1)
    nc_ref[...] = (nseg + (_CH - 1)) // _CH


def _sc_body(R, S_PAD, rpw, nch,
             sidx_hbm, rid_hbm, w_hbm, nc_hbm, nsl_hbm, vr_hbm, out_hbm,
             sg_v, rg_v, wg_v, rows0, rows1, wrow0, wrow1, g0, g1,
             nc_v, nsl_v, acc_v, sums_v, sem0, sem1):
    wid = lax.axis_index("s") * _NC + lax.axis_index("c")
    base = wid * rpw
    pltpu.sync_copy(nc_hbm.at[pl.ds(base, rpw)], nc_v)
    pltpu.sync_copy(nsl_hbm.at[pl.ds(base, rpw)], nsl_v)
    lane16 = lax.iota(jnp.int32, _L)
    zeros = jnp.zeros((_L,), jnp.float32)
    izeros = jnp.zeros((_L,), jnp.int32)
    ngroups = rpw // _L
    rows_b = (rows0, rows1)
    wrow_b = (wrow0, wrow1)
    g_b = (g0, g1)
    sems = (sem0, sem1)

    def prep(k, nsl, nc):
        p_ = k & 1
        for t in range((_PP + _L) // _L):
            rows_b[p_][pl.ds(t * _L, _L)] = izeros

        def zw(s, c):
            for c8 in range(_DL // _L):
                wrow_b[p_][pl.ds(s * _DL + c8 * _L, _L)] = zeros
            return c
        lax.fori_loop(0, nsl, zw, 0)
        rbase = k * S_PAD

        def sc_j(j, c):
            off = rbase + j * _CH
            for v in range(_CH // _L):
                sidx = sg_v[pl.ds(off + v * _L, _L)]
                rid = rg_v[pl.ds(off + v * _L, _L)]
                wv = wg_v[pl.ds(off + v * _L, _L)]
                plsc.addupdate_scatter(wrow_b[p_], [sidx], wv)
                plsc.store_scatter(
                    rows_b[p_], [lax.shift_right_logical(sidx, 7)], rid)
            return c
        lax.fori_loop(0, nc, sc_j, 0)

    def fire(k, nsl):
        p_ = k & 1

        def fc(c, x):
            pltpu.async_copy(
                vr_hbm.at[rows_b[p_].at[pl.ds(c * _L, _L)]],
                g_b[p_].at[pl.ds(c * _L, _L)], sems[p_])
            return x
        lax.fori_loop(0, (nsl + _L - 1) // _L, fc, 0)

    def drain_mac(k, nsl):
        p_ = k & 1

        def dc(c, x):
            pltpu.make_async_copy(
                vr_hbm.at[rows_b[p_].at[pl.ds(c * _L, _L)]],
                g_b[p_].at[pl.ds(c * _L, _L)], sems[p_]).wait()
            return x
        lax.fori_loop(0, (nsl + _L - 1) // _L, dc, 0)

        def mk(s, acc):
            for c8 in range(_DL // _L):
                acc = acc + (g_b[p_][s, pl.ds(c8 * _L, _L)]
                             * wrow_b[p_][pl.ds(s * _DL + c8 * _L, _L)])
            return acc
        acc = lax.fori_loop(0, nsl, mk, zeros)
        acc_v[pl.ds(k * _L, _L)] = acc

    def group_body(g, carry):
        off0 = (base + g * _L) * S_PAD
        gsz = _L * S_PAD
        pltpu.sync_copy(sidx_hbm.at[pl.ds(off0, gsz)], sg_v)
        pltpu.sync_copy(rid_hbm.at[pl.ds(off0, gsz)], rg_v)
        pltpu.sync_copy(w_hbm.at[pl.ds(off0, gsz)], wg_v)
        ncg = jnp.clip(nc_v[pl.ds(g * _L, _L)], 0, nch)
        nslg = jnp.clip(nsl_v[pl.ds(g * _L, _L)], 0, _PP)
        prep(0, nslg[0], ncg[0])
        fire(0, nslg[0])
        for k in range(1, _L):
            prep(k, nslg[k], ncg[k])
            fire(k, nslg[k])
            drain_mac(k - 1, nslg[k - 1])
        drain_mac(_L - 1, nslg[_L - 1])
        tot = zeros
        for j in range(_L):
            tot = tot + plsc.load_gather(acc_v, [lane16 * _L + j])
        sums_v[...] = tot
        pltpu.sync_copy(sums_v, out_hbm.at[pl.ds(base + g * _L, _L)])
        return carry

    lax.fori_loop(0, ngroups, group_body, 0)


def kernel(volume, tvals, M, b, src, dst):
    nx, ny, nz = volume.shape
    R, Sp1 = tvals.shape
    S = Sp1 - 1
    nch = -(-S // _CH)
    S_PAD = nch * _CH
    rpw = R // _NW

    d = dst - src
    ray_len = jnp.sqrt(jnp.sum(d * d, axis=1))
    p = jnp.concatenate(
        [src, d, ray_len[:, None], jnp.zeros((R, 1), jnp.float32)], axis=1)
    mq = M.astype(jnp.bfloat16).astype(jnp.float32)
    mb = jnp.concatenate([mq.reshape(9), b, jnp.zeros((4,), jnp.float32)])

    RB = 256
    grid = (R // RB,)
    sidx, rid, w, nc, nsl = pl.pallas_call(
        functools.partial(_stage1_body, S, S_PAD, nx, ny, nz),
        grid=grid,
        in_specs=[
            pl.BlockSpec((RB, Sp1), lambda i: (i, 0)),
            pl.BlockSpec((RB, 8), lambda i: (i, 0)),
            pl.BlockSpec(memory_space=pltpu.SMEM),
        ],
        out_specs=[
            pl.BlockSpec((RB, S_PAD), lambda i: (i, 0)),
            pl.BlockSpec((RB, S_PAD), lambda i: (i, 0)),
            pl.BlockSpec((RB, S_PAD), lambda i: (i, 0)),
            pl.BlockSpec((RB, 1), lambda i: (i, 0)),
            pl.BlockSpec((RB, 1), lambda i: (i, 0)),
        ],
        out_shape=[
            jax.ShapeDtypeStruct((R, S_PAD), jnp.int32),
            jax.ShapeDtypeStruct((R, S_PAD), jnp.int32),
            jax.ShapeDtypeStruct((R, S_PAD), jnp.float32),
            jax.ShapeDtypeStruct((R, 1), jnp.int32),
            jax.ShapeDtypeStruct((R, 1), jnp.int32),
        ],
    )(tvals, p, mb)

    vr = jnp.transpose(volume, (2, 1, 0)).reshape(-1, _DL)

    mesh = plsc.VectorSubcoreMesh(core_axis_name="c", subcore_axis_name="s")
    sino = pl.kernel(
        functools.partial(_sc_body, R, S_PAD, rpw, nch),
        out_type=jax.ShapeDtypeStruct((R,), jnp.float32),
        mesh=mesh,
        compiler_params=pltpu.CompilerParams(needs_layout_passes=False),
        scratch_types=[
            pltpu.VMEM((_L * S_PAD,), jnp.int32),
            pltpu.VMEM((_L * S_PAD,), jnp.int32),
            pltpu.VMEM((_L * S_PAD,), jnp.float32),
            pltpu.VMEM((_PP + _L,), jnp.int32),
            pltpu.VMEM((_PP + _L,), jnp.int32),
            pltpu.VMEM(((_PP + 1) * _DL,), jnp.float32),
            pltpu.VMEM(((_PP + 1) * _DL,), jnp.float32),
            pltpu.VMEM((_PP, _DL), jnp.float32),
            pltpu.VMEM((_PP, _DL), jnp.float32),
            pltpu.VMEM((rpw,), jnp.int32),
            pltpu.VMEM((rpw,), jnp.int32),
            pltpu.VMEM((_L * _L,), jnp.float32),
            pltpu.VMEM((_L,), jnp.float32),
            pltpu.SemaphoreType.DMA,
            pltpu.SemaphoreType.DMA,
        ],
    )(sidx.reshape(-1), rid.reshape(-1), w.reshape(-1),
      nc.reshape(R), nsl.reshape(R), vr)
    return sino


# speedup vs baseline: 1.2806x; 1.2806x over previous
import functools

import jax
import jax.numpy as jnp
from jax import lax
from jax.experimental import pallas as pl
from jax.experimental.pallas import tpu as pltpu
from jax.experimental.pallas import tpu_sc as plsc

_NC = 2
_NS = 16
_L = 16
_NW = _NC * _NS

_CH = 112


def _stage1_body(S, S_PAD, nx, ny, nz, tv_ref, p_ref, mb_ref, idx_ref, w_ref,
                 nc_ref):
    tv = tv_ref[...]
    p = p_ref[...]
    t0 = tv[:, :-1]
    t1 = tv[:, 1:]
    fin = jnp.isfinite(t0) & jnp.isfinite(t1)
    t0s = jnp.where(fin, t0, 0.0)
    t1s = jnp.where(fin, t1, 0.0)
    valid = fin & (t1 > t0)
    tmid = 0.5 * (t0s + t1s)
    seg = (t1s - t0s) * p[:, 6:7]
    def q(x):
        return x.astype(jnp.bfloat16).astype(jnp.float32)
    px = q(p[:, 0:1] + tmid * p[:, 3:4])
    py = q(p[:, 1:2] + tmid * p[:, 4:5])
    pz = q(p[:, 2:3] + tmid * p[:, 5:6])
    vx = px * mb_ref[0] + py * mb_ref[1] + pz * mb_ref[2] + mb_ref[9]
    vy = px * mb_ref[3] + py * mb_ref[4] + pz * mb_ref[5] + mb_ref[10]
    vz = px * mb_ref[6] + py * mb_ref[7] + pz * mb_ref[8] + mb_ref[11]
    ix = jnp.floor(vx).astype(jnp.int32)
    iy = jnp.floor(vy).astype(jnp.int32)
    iz = jnp.floor(vz).astype(jnp.int32)
    inb = ((ix >= 0) & (ix < nx) & (iy >= 0) & (iy < ny)
           & (iz >= 0) & (iz < nz))
    ixc = jnp.clip(ix, 0, nx - 1)
    iyc = jnp.clip(iy, 0, ny - 1)
    izc = jnp.clip(iz, 0, nz - 1)
    lin = (ixc * ny + iyc) * nz + izc
    w = jnp.where(valid & inb, seg, 0.0)
    rb = tv.shape[0]
    pad = S_PAD - S
    idx_ref[...] = jnp.concatenate(
        [lin, jnp.zeros((rb, pad), jnp.int32)], axis=1)
    w_ref[...] = jnp.concatenate(
        [w, jnp.zeros((rb, pad), jnp.float32)], axis=1)
    nseg = jnp.sum(fin.astype(jnp.int32), axis=1, keepdims=True)
    nc_ref[...] = (nseg + (_CH - 1)) // _CH


def _sc_body(R, S_PAD, rpw, nch,
             idx_hbm, w_hbm, nc_hbm, vol_hbm, out_hbm,
             idx_v0, idx_v1, w_v0, w_v1, g_v0, g_v1,
             nc_v, sums_v, acc_v, sem0, sem1):
    wid = lax.axis_index("s") * _NC + lax.axis_index("c")
    base = wid * rpw
    pltpu.sync_copy(nc_hbm.at[pl.ds(base, rpw)], nc_v)
    lane = lax.iota(jnp.int32, _L)
    ngroups = rpw // _L
    npairs = ngroups // 2

    gsz = _L * S_PAD

    def load(gi, idx_b, w_b):
        off0 = (base + gi * _L) * S_PAD
        pltpu.sync_copy(idx_hbm.at[pl.ds(off0, gsz)], idx_b)
        pltpu.sync_copy(w_hbm.at[pl.ds(off0, gsz)], w_b)

    def fire(gi, idx_b, g_b, sem):
        ncg = nc_v[pl.ds(gi * _L, _L)]
        for r16 in range(_L):
            def fire_j(j, carry):
                off = r16 * S_PAD + j * _CH
                pltpu.async_copy(
                    vol_hbm.at[idx_b.at[pl.ds(off, _CH)]],
                    g_b.at[pl.ds(off, _CH)], sem)
                return carry
            lax.fori_loop(0, ncg[r16], fire_j, 0)

    def drain_mac_store(gi, idx_b, w_b, g_b, sem):
        ncg = nc_v[pl.ds(gi * _L, _L)]
        for r16 in range(_L):
            def wait_j(j, carry):
                off = r16 * S_PAD + j * _CH
                pltpu.make_async_copy(
                    vol_hbm.at[idx_b.at[pl.ds(off, _CH)]],
                    g_b.at[pl.ds(off, _CH)], sem).wait()
                return carry
            lax.fori_loop(0, ncg[r16], wait_j, 0)
        for r16 in range(_L):
            def mac_j(j, acc):
                off = r16 * S_PAD + j * _CH
                for k2 in range(_CH // _L):
                    acc = acc + (g_b[pl.ds(off + k2 * _L, _L)]
                                 * w_b[pl.ds(off + k2 * _L, _L)])
                return acc
            acc = lax.fori_loop(0, ncg[r16], mac_j,
                                jnp.zeros((_L,), jnp.float32))
            acc_v[pl.ds(r16 * _L, _L)] = acc
        tot = jnp.zeros((_L,), jnp.float32)
        for j in range(_L):
            tot = tot + plsc.load_gather(acc_v, [lane * _L + j])
        sums_v[...] = tot
        pltpu.sync_copy(sums_v, out_hbm.at[pl.ds(base + gi * _L, _L)])

    load(0, idx_v0, w_v0)
    fire(0, idx_v0, g_v0, sem0)

    def pair_body(g2, carry):
        g0 = 2 * g2
        g1 = g0 + 1
        load(g1, idx_v1, w_v1)
        fire(g1, idx_v1, g_v1, sem1)
        drain_mac_store(g0, idx_v0, w_v0, g_v0, sem0)

        @pl.when(g2 + 1 < npairs)
        def _():
            load(g0 + 2, idx_v0, w_v0)
            fire(g0 + 2, idx_v0, g_v0, sem0)

        drain_mac_store(g1, idx_v1, w_v1, g_v1, sem1)
        return carry

    lax.fori_loop(0, npairs, pair_body, 0)


def kernel(volume, tvals, M, b, src, dst):
    nx, ny, nz = volume.shape
    R, Sp1 = tvals.shape
    S = Sp1 - 1
    nch = -(-S // _CH)
    S_PAD = nch * _CH
    rpw = R // _NW

    d = dst - src
    ray_len = jnp.sqrt(jnp.sum(d * d, axis=1))
    p = jnp.concatenate(
        [src, d, ray_len[:, None], jnp.zeros((R, 1), jnp.float32)], axis=1)
    mq = M.astype(jnp.bfloat16).astype(jnp.float32)
    mb = jnp.concatenate([mq.reshape(9), b, jnp.zeros((4,), jnp.float32)])

    RB = 256
    grid = (R // RB,)
    idx, w, nc = pl.pallas_call(
        functools.partial(_stage1_body, S, S_PAD, nx, ny, nz),
        grid=grid,
        in_specs=[
            pl.BlockSpec((RB, Sp1), lambda i: (i, 0)),
            pl.BlockSpec((RB, 8), lambda i: (i, 0)),
            pl.BlockSpec(memory_space=pltpu.SMEM),
        ],
        out_specs=[
            pl.BlockSpec((RB, S_PAD), lambda i: (i, 0)),
            pl.BlockSpec((RB, S_PAD), lambda i: (i, 0)),
            pl.BlockSpec((RB, 1), lambda i: (i, 0)),
        ],
        out_shape=[
            jax.ShapeDtypeStruct((R, S_PAD), jnp.int32),
            jax.ShapeDtypeStruct((R, S_PAD), jnp.float32),
            jax.ShapeDtypeStruct((R, 1), jnp.int32),
        ],
    )(tvals, p, mb)

    nc1 = nc.reshape(R)
    vol_flat = volume.reshape(-1)

    mesh = plsc.VectorSubcoreMesh(core_axis_name="c", subcore_axis_name="s")
    sino = pl.kernel(
        functools.partial(_sc_body, R, S_PAD, rpw, nch),
        out_type=jax.ShapeDtypeStruct((R,), jnp.float32),
        mesh=mesh,
        compiler_params=pltpu.CompilerParams(needs_layout_passes=False),
        scratch_types=[
            pltpu.VMEM((_L * S_PAD,), jnp.int32),
            pltpu.VMEM((_L * S_PAD,), jnp.int32),
            pltpu.VMEM((_L * S_PAD,), jnp.float32),
            pltpu.VMEM((_L * S_PAD,), jnp.float32),
            pltpu.VMEM((_L * S_PAD,), jnp.float32),
            pltpu.VMEM((_L * S_PAD,), jnp.float32),
            pltpu.VMEM((rpw,), jnp.int32),
            pltpu.VMEM((_L,), jnp.float32),
            pltpu.VMEM((_L * _L,), jnp.float32),
            pltpu.SemaphoreType.DMA,
            pltpu.SemaphoreType.DMA,
        ],
    )(idx.reshape(-1), w.reshape(-1), nc1, vol_flat)
    return sino
